# baseline (speedup 1.0000x reference)
import functools

import jax
import jax.numpy as jnp
from jax import lax
from jax.experimental import pallas as pl
from jax.experimental.pallas import tpu as pltpu
from jax.experimental.pallas import tpu_sc as plsc

_N = 10000
_NC = 2
_NS = 16
_NW = _NC * _NS
_K = 80
_DEG_PAD = 10240
_NPAD = 10240
_RSTG = 128
_F = 64
_NBUF = 5
_NBUF2 = 5
_K2 = 112

_mesh = functools.partial(
    plsc.VectorSubcoreMesh,
    core_axis_name="c", subcore_axis_name="s",
    num_cores=_NC, num_subcores=_NS)


@functools.lru_cache(maxsize=None)
def _dinv_kernel(nchunk):
    nrow = _DEG_PAD // 16
    rstripe = nrow // _NW

    @functools.partial(
        pl.kernel,
        out_type=jax.ShapeDtypeStruct((_NW, rstripe, 16), jnp.float32),
        mesh=_mesh(),
        scratch_types=[
            pltpu.VMEM((nchunk, _K2), jnp.int32),
            pltpu.VMEM((nrow, 16), jnp.float32),
            pltpu.VMEM((rstripe, 16), jnp.float32),
            pltpu.VMEM((rstripe, 16), jnp.float32),
            pltpu.VMEM_SHARED((_NS, nrow, 16), jnp.float32),
        ],
        compiler_params=pltpu.CompilerParams(
            needs_layout_passes=False, use_tc_tiling_on_sc=False),
    )
    def dinv_k(dst_hbm, out_hbm, dstv, degv, red, tmp, stage):
        cid = lax.axis_index("c")
        sid = lax.axis_index("s")
        wid = sid * _NC + cid
        pltpu.sync_copy(dst_hbm.at[sid], dstv)
        zero16 = jnp.zeros((16,), jnp.float32)

        def zbody(i, carry):
            degv[i, :] = zero16
            return carry

        lax.fori_loop(0, nrow, zbody, 0)
        one16 = jnp.ones((16,), jnp.float32)

        def hbody(i, carry):
            for j in range(_K2 // 16):
                idx = dstv[i, pl.ds(j * 16, 16)]
                plsc.addupdate_scatter(
                    degv,
                    [lax.shift_right_logical(idx, 4),
                     lax.bitwise_and(idx, 15)], one16)
            return carry

        lax.fori_loop(0, nchunk, hbody, 0)
        pltpu.sync_copy(degv, stage.at[sid])
        plsc.subcore_barrier()

        def ibody(i, carry):
            red[i, :] = one16
            return carry

        lax.fori_loop(0, rstripe, ibody, 0)
        r0 = wid * rstripe
        for tt in range(_NS):
            pltpu.sync_copy(
                stage.at[tt, pl.ds(pl.multiple_of(r0, 4), rstripe)], tmp)

            def abody(i, carry):
                red[i, :] = red[i, :] + tmp[i, :]
                return carry

            lax.fori_loop(0, rstripe, abody, 0)

        def rbody(i, carry):
            x = red[i, :]
            bits = plsc.bitcast(x, jnp.int32)
            y = plsc.bitcast(
                0x5F3759DF - lax.shift_right_logical(bits, 1), jnp.float32)
            for _ in range(3):
                y = y * (1.5 - 0.5 * x * y * y)
            red[i, :] = y
            return carry

        lax.fori_loop(0, rstripe, rbody, 0)
        pltpu.sync_copy(red, out_hbm.at[wid])

    return dinv_k


@functools.lru_cache(maxsize=None)
def _prop_kernel(nchunk):
    rpt = _NPAD // _NS

    @functools.partial(
        pl.kernel,
        out_type=jax.ShapeDtypeStruct((_NC, _NPAD, _F), jnp.float32),
        mesh=_mesh(),
        scratch_types=[
            pltpu.VMEM((_NC * nchunk, _K2), jnp.int32),
            pltpu.VMEM((_NC * nchunk, _K2), jnp.int32),
            pltpu.VMEM((_NBUF, _K2, _F), jnp.float32),
            pltpu.VMEM((_RSTG, _F), jnp.float32),
            pltpu.VMEM_SHARED((_NPAD, _F), jnp.float32),
            [pltpu.SemaphoreType.DMA] * _NBUF,
            [pltpu.SemaphoreType.DMA] * _NBUF,
        ],
        compiler_params=pltpu.CompilerParams(
            needs_layout_passes=False, use_tc_tiling_on_sc=False),
    )
    def prop_k(g_hbm, src_hbm, dst_hbm, out_hbm, srcv, dstv, buf, stg, acc,
               gsem, ssem):
        cid = lax.axis_index("c")
        sid = lax.axis_index("s")
        c0 = cid * nchunk
        pltpu.sync_copy(src_hbm.at[sid], srcv)
        pltpu.sync_copy(dst_hbm.at[sid], dstv)

        zero16 = jnp.zeros((16,), jnp.float32)

        def zbody(i, carry):
            for j in range(_F // 16):
                stg[i, pl.ds(j * 16, 16)] = zero16
            return carry

        lax.fori_loop(0, _RSTG, zbody, 0)
        base = sid * rpt
        for q in range(rpt // _RSTG):
            pltpu.sync_copy(stg, acc.at[pl.ds(base + q * _RSTG, _RSTG)])
        plsc.subcore_barrier()

        for b in range(_NBUF):
            pltpu.async_copy(g_hbm.at[srcv.at[c0 + b]], buf.at[b], gsem[b])

        def outer(gidx, carry):
            jbase = c0 + gidx * _NBUF
            for b in range(_NBUF):
                j = jbase + b
                pltpu.make_async_copy(
                    g_hbm.at[srcv.at[j]], buf.at[b], gsem[b]).wait()
                pltpu.async_copy(
                    buf.at[b], acc.at[dstv.at[j]], ssem[b], add=True)
            for b in range(_NBUF):
                j = jbase + b
                pltpu.make_async_copy(
                    buf.at[b], acc.at[dstv.at[j]], ssem[b]).wait()
                nxt = j + _NBUF

                @pl.when(nxt < c0 + nchunk)
                def _():
                    pltpu.async_copy(
                        g_hbm.at[srcv.at[nxt]], buf.at[b], gsem[b])
            return carry

        lax.fori_loop(0, nchunk // _NBUF, outer, 0)
        plsc.subcore_barrier()

        pltpu.sync_copy(acc.at[pl.ds(base, rpt)],
                        out_hbm.at[cid, pl.ds(base, rpt)])

    return prop_k


@functools.lru_cache(maxsize=None)
def _prop2_kernel(nchunk):
    rpt = _NPAD // _NS

    @functools.partial(
        pl.kernel,
        out_type=jax.ShapeDtypeStruct((_NC, _NPAD, _F), jnp.float32),
        mesh=_mesh(),
        scratch_types=[
            pltpu.VMEM((nchunk, _K2), jnp.int32),
            pltpu.VMEM((nchunk, _K2), jnp.int32),
            pltpu.VMEM((_NBUF2, _K2, _F), jnp.float32),
            pltpu.VMEM((_RSTG, _F), jnp.float32),
            pltpu.VMEM_SHARED((_NPAD, _F), jnp.float32),
            [pltpu.SemaphoreType.DMA] * _NBUF2,
            [pltpu.SemaphoreType.DMA] * _NBUF2,
        ],
        compiler_params=pltpu.CompilerParams(
            needs_layout_passes=False, use_tc_tiling_on_sc=False),
    )
    def prop_k(g_hbm, src_hbm, dst_hbm, out_hbm, srcv, dstv, buf, stg, acc,
               gsem, ssem):
        cid = lax.axis_index("c")
        sid = lax.axis_index("s")
        cps = pltpu.async_copy(src_hbm.at[sid], srcv, gsem[0])
        cpd = pltpu.async_copy(dst_hbm.at[sid], dstv, gsem[1])
        gh = g_hbm.at[cid]

        zero16 = jnp.zeros((16,), jnp.float32)

        def zbody(i, carry):
            for j in range(_F // 16):
                stg[i, pl.ds(j * 16, 16)] = zero16
            return carry

        lax.fori_loop(0, _RSTG, zbody, 0)
        base = sid * rpt
        for q in range(rpt // _RSTG):
            pltpu.sync_copy(stg, acc.at[pl.ds(base + q * _RSTG, _RSTG)])
        cps.wait()
        cpd.wait()
        plsc.subcore_barrier()

        for b in range(_NBUF2):
            pltpu.async_copy(gh.at[srcv.at[b]], buf.at[b], gsem[b])

        def outer(gidx, carry):
            jbase = gidx * _NBUF2
            for b in range(_NBUF2):
                j = jbase + b
                pltpu.make_async_copy(
                    gh.at[srcv.at[j]], buf.at[b], gsem[b]).wait()
                pltpu.async_copy(
                    buf.at[b], acc.at[dstv.at[j]], ssem[b], add=True)
            for b in range(_NBUF2):
                j = jbase + b
                pltpu.make_async_copy(
                    buf.at[b], acc.at[dstv.at[j]], ssem[b]).wait()
                nxt = j + _NBUF2

                @pl.when(nxt < nchunk)
                def _():
                    pltpu.async_copy(gh.at[srcv.at[nxt]], buf.at[b], gsem[b])
            return carry

        lax.fori_loop(0, nchunk // _NBUF2, outer, 0)
        plsc.subcore_barrier()

        pltpu.sync_copy(acc.at[pl.ds(base, rpt)],
                        out_hbm.at[cid, pl.ds(base, rpt)])

    return prop_k


def _silu(t):
    return t * (1.0 / (1.0 + jnp.exp(-t)))


_NBLK = 2
_BR = _N // _NBLK


def _tc_prologue(dinv, x, w1):
    def body(dinv_ref, x_ref, w1_ref, g_ref):
        r = dinv_ref[...] * jnp.dot(
            x_ref[...], w1_ref[...], preferred_element_type=jnp.float32)
        g_ref[0] = r[:, :_F]
        g_ref[1] = r[:, _F:]

    return pl.pallas_call(
        body,
        grid=(_NBLK,),
        in_specs=[
            pl.BlockSpec((_BR, 1), lambda i: (i, 0)),
            pl.BlockSpec((_BR, x.shape[1]), lambda i: (i, 0)),
            pl.BlockSpec(w1.shape, lambda i: (0, 0)),
        ],
        out_specs=pl.BlockSpec((2, _BR, _F), lambda i: (0, i, 0)),
        out_shape=jax.ShapeDtypeStruct((2, _N, _F), jnp.float32),
    )(dinv, x, w1)


def _tc_layer(s, g, dinv, b, w, split_out):
    def body(s_ref, g_ref, dinv_ref, b_ref, w_ref, out_ref):
        dv = dinv_ref[...]
        ta = dv * (s_ref[0] + g_ref[0]) + b_ref[:, :_F]
        tb = dv * (s_ref[1] + g_ref[1]) + b_ref[:, _F:]
        r = dv * (
            jnp.dot(_silu(ta), w_ref[:_F], preferred_element_type=jnp.float32)
            + jnp.dot(_silu(tb), w_ref[_F:], preferred_element_type=jnp.float32))
        if split_out:
            out_ref[0] = r[:, :_F]
            out_ref[1] = r[:, _F:]
        else:
            out_ref[...] = r

    if split_out:
        out_spec = pl.BlockSpec((2, _BR, _F), lambda i: (0, i, 0))
        out_shape = jax.ShapeDtypeStruct((2, _N, _F), jnp.float32)
    else:
        out_spec = pl.BlockSpec((_BR, w.shape[1]), lambda i: (i, 0))
        out_shape = jax.ShapeDtypeStruct((_N, w.shape[1]), jnp.float32)
    return pl.pallas_call(
        body,
        grid=(_NBLK,),
        in_specs=[
            pl.BlockSpec((2, _BR, _F), lambda i: (0, i, 0)),
            pl.BlockSpec((2, _BR, _F), lambda i: (0, i, 0)),
            pl.BlockSpec((_BR, 1), lambda i: (i, 0)),
            pl.BlockSpec(b.shape, lambda i: (0, 0)),
            pl.BlockSpec(w.shape, lambda i: (0, 0)),
        ],
        out_specs=out_spec,
        out_shape=out_shape,
    )(s, g, dinv, b, w)


def _tc_final(s, g, dinv, b):
    F = g.shape[1]

    def body(s_ref, g_ref, dinv_ref, b_ref, out_ref):
        t = dinv_ref[...] * (s_ref[0] + s_ref[1] + g_ref[...]) + b_ref[...]
        m = jnp.max(t, axis=1, keepdims=True)
        sh = t - m
        lse = jnp.log(jnp.sum(jnp.exp(sh), axis=1, keepdims=True))
        out_ref[...] = sh - lse

    return pl.pallas_call(
        body,
        grid=(_NBLK,),
        in_specs=[
            pl.BlockSpec((2, _BR, F), lambda i: (0, i, 0)),
            pl.BlockSpec((_BR, F), lambda i: (i, 0)),
            pl.BlockSpec((_BR, 1), lambda i: (i, 0)),
            pl.BlockSpec(b.shape, lambda i: (0, 0)),
        ],
        out_specs=pl.BlockSpec((_BR, F), lambda i: (i, 0)),
        out_shape=jax.ShapeDtypeStruct((_N, F), jnp.float32),
    )(s, g, dinv, b)


def kernel(x, edge_index, W1, b1, W2, b2, W3, b3):
    E = edge_index.shape[1]
    cdiv = _NS * _K2 * _NBUF2 * _NC
    epad = -(-E // cdiv) * cdiv
    npadd = epad - E
    psrc = jnp.concatenate([edge_index[0], jnp.zeros((npadd,), jnp.int32)])
    pdst = jnp.concatenate(
        [edge_index[1],
         _N + (jnp.arange(npadd, dtype=jnp.int32) % (_NPAD - _N))])
    nchunk2 = epad // (_NS * _K2)
    src2 = psrc.reshape(_NS, nchunk2, _K2)
    dst2 = pdst.reshape(_NS, nchunk2, _K2)

    dvp = _dinv_kernel(nchunk2)(dst2)
    dinv = dvp.reshape(_DEG_PAD)[:_N, None]
    g1 = _tc_prologue(dinv, x, W1)
    prop2 = _prop2_kernel(nchunk2)
    s1 = prop2(g1, src2, dst2)
    g2 = _tc_layer(s1, g1, dinv, b1.reshape(1, -1), W2, split_out=True)
    s2 = prop2(g2, src2, dst2)
    g3 = _tc_layer(s2, g2, dinv, b2.reshape(1, -1), W3, split_out=False)
    s3 = _prop_kernel(nchunk2 // _NC)(g3, src2, dst2)
    return _tc_final(s3, g3, dinv, b3.reshape(1, -1))


# speedup vs baseline: 1.4065x; 1.4065x over previous
import functools

import jax
import jax.numpy as jnp
from jax import lax
from jax.experimental import pallas as pl
from jax.experimental.pallas import tpu as pltpu
from jax.experimental.pallas import tpu_sc as plsc

_N = 10000
_NC = 2
_NS = 16
_NW = _NC * _NS
_K = 80
_DEG_PAD = 10240
_NPAD = 10240
_RSTG = 128
_F = 64
_NBUF = 5
_NBUF2 = 5
_K2 = 80

_mesh = functools.partial(
    plsc.VectorSubcoreMesh,
    core_axis_name="c", subcore_axis_name="s",
    num_cores=_NC, num_subcores=_NS)


@functools.lru_cache(maxsize=None)
def _dinv_kernel(nchunk):
    nrow = _DEG_PAD // 16
    rstripe = nrow // _NW

    @functools.partial(
        pl.kernel,
        out_type=jax.ShapeDtypeStruct((_NW, rstripe, 16), jnp.float32),
        mesh=_mesh(),
        scratch_types=[
            pltpu.VMEM((nchunk, _K2), jnp.int32),
            pltpu.VMEM((nrow, 16), jnp.float32),
            pltpu.VMEM((rstripe, 16), jnp.float32),
            pltpu.VMEM((rstripe, 16), jnp.float32),
            pltpu.VMEM_SHARED((_NS, nrow, 16), jnp.float32),
        ],
        compiler_params=pltpu.CompilerParams(
            needs_layout_passes=False, use_tc_tiling_on_sc=False),
    )
    def dinv_k(dst_hbm, out_hbm, dstv, degv, red, tmp, stage):
        cid = lax.axis_index("c")
        sid = lax.axis_index("s")
        wid = sid * _NC + cid
        pltpu.sync_copy(dst_hbm.at[sid], dstv)
        zero16 = jnp.zeros((16,), jnp.float32)

        def zbody(i, carry):
            degv[i, :] = zero16
            return carry

        lax.fori_loop(0, nrow, zbody, 0)
        one16 = jnp.ones((16,), jnp.float32)

        def hbody(i, carry):
            for j in range(_K2 // 16):
                idx = dstv[i, pl.ds(j * 16, 16)]
                plsc.addupdate_scatter(
                    degv,
                    [lax.shift_right_logical(idx, 4),
                     lax.bitwise_and(idx, 15)], one16)
            return carry

        lax.fori_loop(0, nchunk, hbody, 0)
        pltpu.sync_copy(degv, stage.at[sid])
        plsc.subcore_barrier()

        def ibody(i, carry):
            red[i, :] = one16
            return carry

        lax.fori_loop(0, rstripe, ibody, 0)
        r0 = wid * rstripe
        for tt in range(_NS):
            pltpu.sync_copy(
                stage.at[tt, pl.ds(pl.multiple_of(r0, 4), rstripe)], tmp)

            def abody(i, carry):
                red[i, :] = red[i, :] + tmp[i, :]
                return carry

            lax.fori_loop(0, rstripe, abody, 0)

        def rbody(i, carry):
            x = red[i, :]
            bits = plsc.bitcast(x, jnp.int32)
            y = plsc.bitcast(
                0x5F3759DF - lax.shift_right_logical(bits, 1), jnp.float32)
            for _ in range(3):
                y = y * (1.5 - 0.5 * x * y * y)
            red[i, :] = y
            return carry

        lax.fori_loop(0, rstripe, rbody, 0)
        pltpu.sync_copy(red, out_hbm.at[wid])

    return dinv_k


@functools.lru_cache(maxsize=None)
def _prop_kernel(nchunk):
    rpt = _NPAD // _NS

    @functools.partial(
        pl.kernel,
        out_type=jax.ShapeDtypeStruct((_NC, _NPAD, _F), jnp.float32),
        mesh=_mesh(),
        scratch_types=[
            pltpu.VMEM((_NC * nchunk, _K2), jnp.int32),
            pltpu.VMEM((_NC * nchunk, _K2), jnp.int32),
            pltpu.VMEM((_NBUF, _K2, _F), jnp.float32),
            pltpu.VMEM((_RSTG, _F), jnp.float32),
            pltpu.VMEM_SHARED((_NPAD, _F), jnp.float32),
            [pltpu.SemaphoreType.DMA] * _NBUF,
            [pltpu.SemaphoreType.DMA] * _NBUF,
        ],
        compiler_params=pltpu.CompilerParams(
            needs_layout_passes=False, use_tc_tiling_on_sc=False),
    )
    def prop_k(g_hbm, src_hbm, dst_hbm, out_hbm, srcv, dstv, buf, stg, acc,
               gsem, ssem):
        cid = lax.axis_index("c")
        sid = lax.axis_index("s")
        c0 = cid * nchunk
        pltpu.sync_copy(src_hbm.at[sid], srcv)
        pltpu.sync_copy(dst_hbm.at[sid], dstv)

        zero16 = jnp.zeros((16,), jnp.float32)

        def zbody(i, carry):
            for j in range(_F // 16):
                stg[i, pl.ds(j * 16, 16)] = zero16
            return carry

        lax.fori_loop(0, _RSTG, zbody, 0)
        base = sid * rpt
        for q in range(rpt // _RSTG):
            pltpu.sync_copy(stg, acc.at[pl.ds(base + q * _RSTG, _RSTG)])
        plsc.subcore_barrier()

        for b in range(_NBUF):
            pltpu.async_copy(g_hbm.at[srcv.at[c0 + b]], buf.at[b], gsem[b])

        def outer(gidx, carry):
            jbase = c0 + gidx * _NBUF
            for b in range(_NBUF):
                j = jbase + b
                pltpu.make_async_copy(
                    g_hbm.at[srcv.at[j]], buf.at[b], gsem[b]).wait()
                pltpu.async_copy(
                    buf.at[b], acc.at[dstv.at[j]], ssem[b], add=True)
            for b in range(_NBUF):
                j = jbase + b
                pltpu.make_async_copy(
                    buf.at[b], acc.at[dstv.at[j]], ssem[b]).wait()
                nxt = j + _NBUF

                @pl.when(nxt < c0 + nchunk)
                def _():
                    pltpu.async_copy(
                        g_hbm.at[srcv.at[nxt]], buf.at[b], gsem[b])
            return carry

        lax.fori_loop(0, nchunk // _NBUF, outer, 0)
        plsc.subcore_barrier()

        pltpu.sync_copy(acc.at[pl.ds(base, rpt)],
                        out_hbm.at[cid, pl.ds(base, rpt)])

    return prop_k


@functools.lru_cache(maxsize=None)
def _prop2_kernel(nchunk):
    rpt = _NPAD // _NS

    @functools.partial(
        pl.kernel,
        out_type=jax.ShapeDtypeStruct((_NC, _NPAD, _F), jnp.float32),
        mesh=_mesh(),
        scratch_types=[
            pltpu.VMEM((nchunk, _K2), jnp.int32),
            pltpu.VMEM((nchunk, _K2), jnp.int32),
            pltpu.VMEM((_NBUF2, _K2, _F), jnp.float32),
            pltpu.VMEM((_RSTG, _F), jnp.float32),
            pltpu.VMEM_SHARED((_NPAD, _F), jnp.float32),
            [pltpu.SemaphoreType.DMA] * _NBUF2,
            [pltpu.SemaphoreType.DMA] * _NBUF2,
        ],
        compiler_params=pltpu.CompilerParams(
            needs_layout_passes=False, use_tc_tiling_on_sc=False),
    )
    def prop_k(g_hbm, src_hbm, dst_hbm, out_hbm, srcv, dstv, buf, stg, acc,
               gsem, ssem):
        cid = lax.axis_index("c")
        sid = lax.axis_index("s")
        cps = pltpu.async_copy(src_hbm.at[sid], srcv, gsem[0])
        cpd = pltpu.async_copy(dst_hbm.at[sid], dstv, gsem[1])
        gh = g_hbm.at[cid]

        zero16 = jnp.zeros((16,), jnp.float32)

        def zbody(i, carry):
            for j in range(_F // 16):
                stg[i, pl.ds(j * 16, 16)] = zero16
            return carry

        lax.fori_loop(0, _RSTG, zbody, 0)
        base = sid * rpt
        for q in range(rpt // _RSTG):
            pltpu.sync_copy(stg, acc.at[pl.ds(base + q * _RSTG, _RSTG)])
        cps.wait()
        cpd.wait()
        plsc.subcore_barrier()

        for b in range(_NBUF2):
            pltpu.async_copy(gh.at[srcv.at[b]], buf.at[b], gsem[b])

        def outer(gidx, carry):
            jbase = gidx * _NBUF2
            for b in range(_NBUF2):
                j = jbase + b
                pltpu.make_async_copy(
                    gh.at[srcv.at[j]], buf.at[b], gsem[b]).wait()
                pltpu.async_copy(
                    buf.at[b], acc.at[dstv.at[j]], ssem[b], add=True)
            for b in range(_NBUF2):
                j = jbase + b
                pltpu.make_async_copy(
                    buf.at[b], acc.at[dstv.at[j]], ssem[b]).wait()
                nxt = j + _NBUF2

                @pl.when(nxt < nchunk)
                def _():
                    pltpu.async_copy(gh.at[srcv.at[nxt]], buf.at[b], gsem[b])
            return carry

        lax.fori_loop(0, nchunk // _NBUF2, outer, 0)
        plsc.subcore_barrier()

        pltpu.sync_copy(acc.at[pl.ds(base, rpt)],
                        out_hbm.at[cid, pl.ds(base, rpt)])

    return prop_k


def _silu(t):
    return t * (1.0 / (1.0 + jnp.exp(-t)))


_NBLK = 2
_BR = _N // _NBLK


def _tc_prologue(dinv, x, w1):
    def body(dinv_ref, x_ref, w1_ref, g_ref):
        r = dinv_ref[...] * jnp.dot(
            x_ref[...], w1_ref[...], preferred_element_type=jnp.float32)
        g_ref[0] = r[:, :_F]
        g_ref[1] = r[:, _F:]

    return pl.pallas_call(
        body,
        grid=(_NBLK,),
        in_specs=[
            pl.BlockSpec((_BR, 1), lambda i: (i, 0)),
            pl.BlockSpec((_BR, x.shape[1]), lambda i: (i, 0)),
            pl.BlockSpec(w1.shape, lambda i: (0, 0)),
        ],
        out_specs=pl.BlockSpec((2, _BR, _F), lambda i: (0, i, 0)),
        out_shape=jax.ShapeDtypeStruct((2, _N, _F), jnp.float32),
    )(dinv, x, w1)


def _tc_layer(s, g, dinv, b, w, split_out):
    def body(s_ref, g_ref, dinv_ref, b_ref, w_ref, out_ref):
        dv = dinv_ref[...]
        ta = dv * (s_ref[0] + g_ref[0]) + b_ref[:, :_F]
        tb = dv * (s_ref[1] + g_ref[1]) + b_ref[:, _F:]
        r = dv * (
            jnp.dot(_silu(ta), w_ref[:_F], preferred_element_type=jnp.float32)
            + jnp.dot(_silu(tb), w_ref[_F:], preferred_element_type=jnp.float32))
        if split_out:
            out_ref[0] = r[:, :_F]
            out_ref[1] = r[:, _F:]
        else:
            out_ref[...] = r

    if split_out:
        out_spec = pl.BlockSpec((2, _BR, _F), lambda i: (0, i, 0))
        out_shape = jax.ShapeDtypeStruct((2, _N, _F), jnp.float32)
    else:
        out_spec = pl.BlockSpec((_BR, w.shape[1]), lambda i: (i, 0))
        out_shape = jax.ShapeDtypeStruct((_N, w.shape[1]), jnp.float32)
    return pl.pallas_call(
        body,
        grid=(_NBLK,),
        in_specs=[
            pl.BlockSpec((2, _BR, _F), lambda i: (0, i, 0)),
            pl.BlockSpec((2, _BR, _F), lambda i: (0, i, 0)),
            pl.BlockSpec((_BR, 1), lambda i: (i, 0)),
            pl.BlockSpec(b.shape, lambda i: (0, 0)),
            pl.BlockSpec(w.shape, lambda i: (0, 0)),
        ],
        out_specs=out_spec,
        out_shape=out_shape,
    )(s, g, dinv, b, w)


def _tc_final(s, g, dinv, b):
    F = g.shape[1]

    def body(s_ref, g_ref, dinv_ref, b_ref, out_ref):
        t = dinv_ref[...] * (s_ref[0] + s_ref[1] + g_ref[...]) + b_ref[...]
        m = jnp.max(t, axis=1, keepdims=True)
        sh = t - m
        lse = jnp.log(jnp.sum(jnp.exp(sh), axis=1, keepdims=True))
        out_ref[...] = sh - lse

    return pl.pallas_call(
        body,
        grid=(_NBLK,),
        in_specs=[
            pl.BlockSpec((2, _BR, F), lambda i: (0, i, 0)),
            pl.BlockSpec((_BR, F), lambda i: (i, 0)),
            pl.BlockSpec((_BR, 1), lambda i: (i, 0)),
            pl.BlockSpec(b.shape, lambda i: (0, 0)),
        ],
        out_specs=pl.BlockSpec((_BR, F), lambda i: (i, 0)),
        out_shape=jax.ShapeDtypeStruct((_N, F), jnp.float32),
    )(s, g, dinv, b)


def kernel(x, edge_index, W1, b1, W2, b2, W3, b3):
    E = edge_index.shape[1]
    cdiv = _NS * _K2 * _NBUF2 * _NC
    epad = -(-E // cdiv) * cdiv
    npadd = epad - E
    psrc = jnp.concatenate([edge_index[0], jnp.zeros((npadd,), jnp.int32)])
    pdst = jnp.concatenate(
        [edge_index[1],
         _N + (jnp.arange(npadd, dtype=jnp.int32) % (_NPAD - _N))])
    nchunk2 = epad // (_NS * _K2)
    src2 = psrc.reshape(_NS, nchunk2, _K2)
    dst2 = pdst.reshape(_NS, nchunk2, _K2)

    dvp = _dinv_kernel(nchunk2)(dst2)
    dinv = dvp.reshape(_DEG_PAD)[:_N, None]
    g1 = _tc_prologue(dinv, x, W1)
    prop2 = _prop2_kernel(nchunk2)
    s1 = prop2(g1, src2, dst2)
    g2 = _tc_layer(s1, g1, dinv, b1.reshape(1, -1), W2, split_out=True)
    s2 = prop2(g2, src2, dst2)
    g3 = _tc_layer(s2, g2, dinv, b2.reshape(1, -1), W3, split_out=False)
    s3 = _prop_kernel(nchunk2 // _NC)(g3, src2, dst2)
    return _tc_final(s3, g3, dinv, b3.reshape(1, -1))
